# Initial kernel scaffold; baseline (speedup 1.0000x reference)
import jax
import jax.numpy as jnp
from jax.experimental import pallas as pl


def kernel(src_feat, dst_feat, edge_feat, edge_index, W, b):
    raise NotImplementedError("write your pallas kernel here")


# speedup vs baseline: 8.7561x; 8.7561x over previous
import functools

import jax
import jax.numpy as jnp
from jax import lax
from jax.experimental import pallas as pl
from jax.experimental.pallas import tpu as pltpu
from jax.experimental.pallas import tpu_sc as plsc

_NC = 2
_NS = 16
_L = 16


def _sc_segment_sums(src_feat, edge_feat, src_idx, dst_idx):
    n, d = src_feat.shape
    e = src_idx.shape[0]
    assert d == 128 and n % _NS == 0 and n % 8 == 0
    ept = e // _NS
    K = 128
    n_main = ept // K
    tail = ept - n_main * K
    assert tail % 8 == 0 and e % _NS == 0
    rpt = (n // (8 * _NS)) * 8
    rem = n - _NS * rpt

    mesh = plsc.VectorSubcoreMesh(
        core_axis_name="c", subcore_axis_name="s",
        num_cores=_NC, num_subcores=_NS)

    @functools.partial(
        pl.kernel,
        out_type=[
            jax.ShapeDtypeStruct((n, d), jnp.float32),
            jax.ShapeDtypeStruct((n, d), jnp.float32),
            jax.ShapeDtypeStruct((n,), jnp.float32),
        ],
        mesh=mesh,
        scratch_types=[
            pltpu.VMEM_SHARED((n, d), jnp.float32),
            pltpu.VMEM_SHARED((n,), jnp.float32),
            pltpu.VMEM((K,), jnp.int32),
            pltpu.VMEM((K,), jnp.int32),
            pltpu.VMEM((K, d), jnp.float32),
            pltpu.VMEM((K,), jnp.float32),
            pltpu.VMEM((K,), jnp.float32),
            pltpu.VMEM((tail,), jnp.int32),
            pltpu.VMEM((tail,), jnp.int32),
            pltpu.SemaphoreType.DMA,
        ],
    )
    def sc_kernel(src_hbm, edge_hbm, sidx_hbm, didx_hbm,
                  out_a, out_b, out_deg,
                  acc, deg_acc, idx_s, idx_d, rows, ones, zvec,
                  idx_s2, idx_d2, sem):
        c = lax.axis_index("c")
        s = lax.axis_index("s")
        zeros16 = jnp.zeros((_L,), jnp.float32)
        ones16 = jnp.full((_L,), 1.0, jnp.float32)
        base_rows = s * rpt

        def zrow(r, _):
            for j in range(d // _L):
                rows[r, pl.ds(j * _L, _L)] = zeros16
            return _
        lax.fori_loop(0, K, zrow, None)

        def vec_init(j, _):
            ones[pl.ds(j * _L, _L)] = ones16
            zvec[pl.ds(j * _L, _L)] = zeros16
            return _
        lax.fori_loop(0, K // _L, vec_init, None)

        def zero_slices(row0, cnt):
            off = 0
            while off < cnt:
                step = min(K, cnt - off)
                pltpu.sync_copy(rows.at[pl.ds(0, step)],
                                acc.at[pl.ds(row0 + off, step)])
                pltpu.sync_copy(zvec.at[pl.ds(0, step)],
                                deg_acc.at[pl.ds(row0 + off, step)])
                off += step

        zero_slices(base_rows, rpt)
        if rem:
            @pl.when(s == _NS - 1)
            def _():
                zero_slices(_NS * rpt, rem)

        plsc.subcore_barrier()

        tile_base = s * ept

        def chunk(g, _):
            base = tile_base + g * K
            pltpu.sync_copy(didx_hbm.at[pl.ds(base, K)], idx_d)

            @pl.when(c == 0)
            def _():
                pltpu.sync_copy(sidx_hbm.at[pl.ds(base, K)], idx_s)
                pltpu.async_copy(src_hbm.at[idx_s], rows, sem).wait()
                pltpu.sync_copy(rows, acc.at[idx_d], add=True)

            @pl.when(c == 1)
            def _():
                pltpu.sync_copy(edge_hbm.at[pl.ds(base, K)], rows)
                pltpu.sync_copy(rows, acc.at[idx_d], add=True)
                pltpu.sync_copy(ones, deg_acc.at[idx_d], add=True)
            return _
        lax.fori_loop(0, n_main, chunk, None)

        if tail:
            base = tile_base + n_main * K
            tail_rows = rows.at[pl.ds(0, tail)]
            pltpu.sync_copy(didx_hbm.at[pl.ds(base, tail)], idx_d2)

            @pl.when(c == 0)
            def _():
                pltpu.sync_copy(sidx_hbm.at[pl.ds(base, tail)], idx_s2)
                pltpu.async_copy(src_hbm.at[idx_s2], tail_rows, sem).wait()
                pltpu.sync_copy(tail_rows, acc.at[idx_d2], add=True)

            @pl.when(c == 1)
            def _():
                pltpu.sync_copy(edge_hbm.at[pl.ds(base, tail)], tail_rows)
                pltpu.sync_copy(tail_rows, acc.at[idx_d2], add=True)
                pltpu.sync_copy(ones.at[pl.ds(0, tail)],
                                deg_acc.at[idx_d2], add=True)

        plsc.subcore_barrier()

        def drain_acc(dst_hbm, row0, cnt):
            off = 0
            while off < cnt:
                step = min(K, cnt - off)
                sl = pl.ds(row0 + off, step)
                pltpu.sync_copy(acc.at[sl], rows.at[pl.ds(0, step)])
                pltpu.sync_copy(rows.at[pl.ds(0, step)], dst_hbm.at[sl])
                off += step

        def drain_deg(row0, cnt):
            off = 0
            while off < cnt:
                step = min(K, cnt - off)
                sl = pl.ds(row0 + off, step)
                pltpu.sync_copy(deg_acc.at[sl], zvec.at[pl.ds(0, step)])
                pltpu.sync_copy(zvec.at[pl.ds(0, step)], out_deg.at[sl])
                off += step

        @pl.when(c == 0)
        def _():
            drain_acc(out_a, base_rows, rpt)
            if rem:
                @pl.when(s == _NS - 1)
                def _():
                    drain_acc(out_a, _NS * rpt, rem)

        @pl.when(c == 1)
        def _():
            drain_acc(out_b, base_rows, rpt)
            drain_deg(base_rows, rpt)
            if rem:
                @pl.when(s == _NS - 1)
                def _():
                    drain_acc(out_b, _NS * rpt, rem)
                    drain_deg(_NS * rpt, rem)

    return sc_kernel(src_feat, edge_feat, src_idx, dst_idx)


def _tc_finish(sa, sb, deg, w1, w2, bias):
    n, d = sa.shape
    bn = 1000
    assert n % bn == 0

    def body(a_ref, b_ref, deg_ref, w1_ref, w2_ref, bias_ref, out_ref):
        acc = jnp.dot(a_ref[...], w1_ref[...], preferred_element_type=jnp.float32)
        acc = acc + jnp.dot(b_ref[...], w2_ref[...], preferred_element_type=jnp.float32)
        dv = deg_ref[...]
        norm = jnp.where(dv > 0.0, lax.rsqrt(dv), 0.0)
        out_ref[...] = acc * norm + bias_ref[...]

    return pl.pallas_call(
        body,
        grid=(n // bn,),
        in_specs=[
            pl.BlockSpec((bn, d), lambda i: (i, 0)),
            pl.BlockSpec((bn, d), lambda i: (i, 0)),
            pl.BlockSpec((bn, 1), lambda i: (i, 0)),
            pl.BlockSpec((d, d), lambda i: (0, 0)),
            pl.BlockSpec((d, d), lambda i: (0, 0)),
            pl.BlockSpec((1, d), lambda i: (0, 0)),
        ],
        out_specs=pl.BlockSpec((bn, d), lambda i: (i, 0)),
        out_shape=jax.ShapeDtypeStruct((n, d), jnp.float32),
    )(sa, sb, deg, w1, w2, bias)


def kernel(src_feat, dst_feat, edge_feat, edge_index, W, b):
    del dst_feat
    n, d = src_feat.shape
    src_idx = edge_index[0]
    dst_idx = edge_index[1]
    sa, sb, deg = _sc_segment_sums(src_feat, edge_feat, src_idx, dst_idx)
    return _tc_finish(sa, sb, deg.reshape(n, 1), W[:d], W[d:], b.reshape(1, d))


# speedup vs baseline: 13.7700x; 1.5726x over previous
import functools

import jax
import jax.numpy as jnp
from jax import lax
from jax.experimental import pallas as pl
from jax.experimental.pallas import tpu as pltpu
from jax.experimental.pallas import tpu_sc as plsc

_NC = 2
_NS = 16
_L = 16


def _sc_segment_sums(src_feat, edge_feat, src_idx, dst_idx):
    n, d = src_feat.shape
    e = src_idx.shape[0]
    assert d == 128 and n % _NS == 0 and n % 8 == 0
    ept = e // _NS
    K = 128
    n_main = ept // K
    tail = ept - n_main * K
    assert tail % 8 == 0 and e % _NS == 0
    rpt = (n // (8 * _NS)) * 8
    rem = n - _NS * rpt

    mesh = plsc.VectorSubcoreMesh(
        core_axis_name="c", subcore_axis_name="s",
        num_cores=_NC, num_subcores=_NS)

    @functools.partial(
        pl.kernel,
        out_type=[
            jax.ShapeDtypeStruct((n, d), jnp.float32),
            jax.ShapeDtypeStruct((n, d), jnp.float32),
            jax.ShapeDtypeStruct((n,), jnp.float32),
        ],
        mesh=mesh,
    scratch_types=[
            pltpu.VMEM_SHARED((n, d), jnp.float32),
            pltpu.VMEM_SHARED((n,), jnp.float32),
            pltpu.VMEM((K,), jnp.int32),
            pltpu.VMEM((K,), jnp.int32),
            pltpu.VMEM((K,), jnp.int32),
            pltpu.VMEM((K,), jnp.int32),
            pltpu.VMEM((K, d), jnp.float32),
            pltpu.VMEM((K, d), jnp.float32),
            pltpu.VMEM((K,), jnp.float32),
            pltpu.VMEM((K,), jnp.float32),
            pltpu.VMEM((tail,), jnp.int32),
            pltpu.VMEM((tail,), jnp.int32),
            pltpu.SemaphoreType.DMA,
            pltpu.SemaphoreType.DMA,
            pltpu.SemaphoreType.DMA,
            pltpu.SemaphoreType.DMA,
            pltpu.SemaphoreType.DMA,
            pltpu.SemaphoreType.DMA,
        ],
    )
    def sc_kernel(src_hbm, edge_hbm, sidx_hbm, didx_hbm,
                  out_a, out_b, out_deg,
                  acc, deg_acc, idx_s0, idx_s1, idx_d0, idx_d1,
                  rows0, rows1, ones, zvec,
                  idx_s2, idx_d2,
                  sem_g0, sem_g1, sem_s0, sem_s1, sem_h0, sem_h1):
        c = lax.axis_index("c")
        s = lax.axis_index("s")
        zeros16 = jnp.zeros((_L,), jnp.float32)
        ones16 = jnp.full((_L,), 1.0, jnp.float32)
        base_rows = s * rpt
        idx_s = [idx_s0, idx_s1]
        idx_d = [idx_d0, idx_d1]
        rows = [rows0, rows1]
        sem_g = [sem_g0, sem_g1]
        sem_s = [sem_s0, sem_s1]
        sem_h = [sem_h0, sem_h1]

        def zrow(r, _):
            for j in range(d // _L):
                rows0[r, pl.ds(j * _L, _L)] = zeros16
            return _
        lax.fori_loop(0, K, zrow, None)

        def vec_init(j, _):
            ones[pl.ds(j * _L, _L)] = ones16
            zvec[pl.ds(j * _L, _L)] = zeros16
            return _
        lax.fori_loop(0, K // _L, vec_init, None)

        def zero_slices(row0, cnt):
            off = 0
            while off < cnt:
                step = min(K, cnt - off)
                pltpu.sync_copy(rows0.at[pl.ds(0, step)],
                                acc.at[pl.ds(row0 + off, step)])
                pltpu.sync_copy(zvec.at[pl.ds(0, step)],
                                deg_acc.at[pl.ds(row0 + off, step)])
                off += step

        zero_slices(base_rows, rpt)
        if rem:
            @pl.when(s == _NS - 1)
            def _():
                zero_slices(_NS * rpt, rem)

        plsc.subcore_barrier()

        tile_base = s * ept
        assert n_main % 2 == 0

        @pl.when(c == 0)
        def _():
            def load_idx(g, p):
                base = tile_base + g * K
                pltpu.sync_copy(sidx_hbm.at[pl.ds(base, K)], idx_s[p])
                pltpu.sync_copy(didx_hbm.at[pl.ds(base, K)], idx_d[p])

            def start_gather(p):
                pltpu.async_copy(src_hbm.at[idx_s[p]], rows[p], sem_g[p])

            def wait_gather(p):
                pltpu.make_async_copy(src_hbm.at[idx_s[p]], rows[p],
                                      sem_g[p]).wait()

            def start_scatter(p):
                pltpu.async_copy(rows[p], acc.at[idx_d[p]], sem_s[p],
                                 add=True)

            def wait_scatter(p):
                pltpu.make_async_copy(rows[p], acc.at[idx_d[p]],
                                      sem_s[p]).wait()

            def body(g, p, q):
                wait_gather(p)
                start_scatter(p)

                @pl.when(g + 1 < n_main)
                def _():
                    start_gather(q)
                wait_scatter(p)

                @pl.when(g + 2 < n_main)
                def _():
                    load_idx(g + 2, p)

            load_idx(0, 0)
            start_gather(0)
            load_idx(1, 1)

            def looped(t, _):
                body(2 * t, 0, 1)
                body(2 * t + 1, 1, 0)
                return _
            lax.fori_loop(0, n_main // 2, looped, None)

        @pl.when(c == 1)
        def _():
            def load_idx(g, p):
                base = tile_base + g * K
                pltpu.sync_copy(didx_hbm.at[pl.ds(base, K)], idx_d[p])

            def start_read(g, p):
                base = tile_base + g * K
                pltpu.async_copy(edge_hbm.at[pl.ds(base, K)], rows[p],
                                 sem_g[p])

            def wait_read(g, p):
                base = tile_base + g * K
                pltpu.make_async_copy(edge_hbm.at[pl.ds(base, K)], rows[p],
                                      sem_g[p]).wait()

            def start_scatter(p):
                pltpu.async_copy(rows[p], acc.at[idx_d[p]], sem_s[p],
                                 add=True)
                pltpu.async_copy(ones, deg_acc.at[idx_d[p]], sem_h[p],
                                 add=True)

            def wait_scatter(p):
                pltpu.make_async_copy(rows[p], acc.at[idx_d[p]],
                                      sem_s[p]).wait()
                pltpu.make_async_copy(ones, deg_acc.at[idx_d[p]],
                                      sem_h[p]).wait()

            def body(g, p, q):
                wait_read(g, p)
                start_scatter(p)

                @pl.when(g + 1 < n_main)
                def _():
                    start_read(g + 1, q)
                wait_scatter(p)

                @pl.when(g + 2 < n_main)
                def _():
                    load_idx(g + 2, p)

            load_idx(0, 0)
            start_read(0, 0)
            load_idx(1, 1)

            def looped(t, _):
                body(2 * t, 0, 1)
                body(2 * t + 1, 1, 0)
                return _
            lax.fori_loop(0, n_main // 2, looped, None)

        if tail:
            base = tile_base + n_main * K
            tail_rows = rows0.at[pl.ds(0, tail)]
            pltpu.sync_copy(didx_hbm.at[pl.ds(base, tail)], idx_d2)

            @pl.when(c == 0)
            def _():
                pltpu.sync_copy(sidx_hbm.at[pl.ds(base, tail)], idx_s2)
                pltpu.async_copy(src_hbm.at[idx_s2], tail_rows,
                                 sem_g0).wait()
                pltpu.sync_copy(tail_rows, acc.at[idx_d2], add=True)

            @pl.when(c == 1)
            def _():
                pltpu.sync_copy(edge_hbm.at[pl.ds(base, tail)], tail_rows)
                pltpu.sync_copy(tail_rows, acc.at[idx_d2], add=True)
                pltpu.sync_copy(ones.at[pl.ds(0, tail)],
                                deg_acc.at[idx_d2], add=True)

        plsc.subcore_barrier()

        def drain_acc(dst_hbm, row0, cnt):
            off = 0
            while off < cnt:
                step = min(K, cnt - off)
                sl = pl.ds(row0 + off, step)
                pltpu.sync_copy(acc.at[sl], rows0.at[pl.ds(0, step)])
                pltpu.sync_copy(rows0.at[pl.ds(0, step)], dst_hbm.at[sl])
                off += step

        def drain_deg(row0, cnt):
            off = 0
            while off < cnt:
                step = min(K, cnt - off)
                sl = pl.ds(row0 + off, step)
                pltpu.sync_copy(deg_acc.at[sl], zvec.at[pl.ds(0, step)])
                pltpu.sync_copy(zvec.at[pl.ds(0, step)], out_deg.at[sl])
                off += step

        @pl.when(c == 0)
        def _():
            drain_acc(out_a, base_rows, rpt)
            if rem:
                @pl.when(s == _NS - 1)
                def _():
                    drain_acc(out_a, _NS * rpt, rem)

        @pl.when(c == 1)
        def _():
            drain_acc(out_b, base_rows, rpt)
            drain_deg(base_rows, rpt)
            if rem:
                @pl.when(s == _NS - 1)
                def _():
                    drain_acc(out_b, _NS * rpt, rem)
                    drain_deg(_NS * rpt, rem)

    return sc_kernel(src_feat, edge_feat, src_idx, dst_idx)


def _tc_finish(sa, sb, deg, w1, w2, bias):
    n, d = sa.shape
    bn = 1000
    assert n % bn == 0

    def body(a_ref, b_ref, deg_ref, w1_ref, w2_ref, bias_ref, out_ref):
        acc = jnp.dot(a_ref[...], w1_ref[...], preferred_element_type=jnp.float32)
        acc = acc + jnp.dot(b_ref[...], w2_ref[...], preferred_element_type=jnp.float32)
        dv = deg_ref[...]
        norm = jnp.where(dv > 0.0, lax.rsqrt(dv), 0.0)
        out_ref[...] = acc * norm + bias_ref[...]

    return pl.pallas_call(
        body,
        grid=(n // bn,),
        in_specs=[
            pl.BlockSpec((bn, d), lambda i: (i, 0)),
            pl.BlockSpec((bn, d), lambda i: (i, 0)),
            pl.BlockSpec((bn, 1), lambda i: (i, 0)),
            pl.BlockSpec((d, d), lambda i: (0, 0)),
            pl.BlockSpec((d, d), lambda i: (0, 0)),
            pl.BlockSpec((1, d), lambda i: (0, 0)),
        ],
        out_specs=pl.BlockSpec((bn, d), lambda i: (i, 0)),
        out_shape=jax.ShapeDtypeStruct((n, d), jnp.float32),
    )(sa, sb, deg, w1, w2, bias)


def kernel(src_feat, dst_feat, edge_feat, edge_index, W, b):
    del dst_feat
    n, d = src_feat.shape
    src_idx = edge_index[0]
    dst_idx = edge_index[1]
    sa, sb, deg = _sc_segment_sums(src_feat, edge_feat, src_idx, dst_idx)
    return _tc_finish(sa, sb, deg.reshape(n, 1), W[:d], W[d:], b.reshape(1, d))


# speedup vs baseline: 14.2428x; 1.0343x over previous
import functools

import jax
import jax.numpy as jnp
from jax import lax
from jax.experimental import pallas as pl
from jax.experimental.pallas import tpu as pltpu
from jax.experimental.pallas import tpu_sc as plsc

_NC = 2
_NS = 16
_L = 16


def _sc_segment_sums(src_feat, edge_feat, src_idx, dst_idx):
    n, d = src_feat.shape
    e = src_idx.shape[0]
    assert d == 128 and n % _NS == 0 and n % 8 == 0
    ept = e // _NS
    K = 128
    n_main = ept // K
    tail = ept - n_main * K
    assert tail % 8 == 0 and e % _NS == 0
    G = 6
    assert n_main % G == 0 and n_main % 2 == 0
    n_blocks = n_main // G
    rpt = (n // (8 * _NS)) * 8
    rem = n - _NS * rpt

    mesh = plsc.VectorSubcoreMesh(
        core_axis_name="c", subcore_axis_name="s",
        num_cores=_NC, num_subcores=_NS)

    @functools.partial(
        pl.kernel,
        out_type=[
            jax.ShapeDtypeStruct((n, d), jnp.float32),
            jax.ShapeDtypeStruct((n, d), jnp.float32),
            jax.ShapeDtypeStruct((n,), jnp.float32),
        ],
        mesh=mesh,
    scratch_types=[
            pltpu.VMEM_SHARED((n, d), jnp.float32),
            pltpu.VMEM_SHARED((n,), jnp.float32),
            pltpu.VMEM((K,), jnp.int32),
            pltpu.VMEM((K,), jnp.int32),
            pltpu.VMEM((K,), jnp.int32),
            pltpu.VMEM((K,), jnp.int32),
            pltpu.VMEM((K, d), jnp.float32),
            pltpu.VMEM((K, d), jnp.float32),
            pltpu.VMEM((K,), jnp.float32),
            pltpu.VMEM((K,), jnp.float32),
            pltpu.VMEM((G * K,), jnp.int32),
            pltpu.VMEM((G * K,), jnp.int32),
            pltpu.VMEM((tail,), jnp.int32),
            pltpu.VMEM((tail,), jnp.int32),
            pltpu.SemaphoreType.DMA,
            pltpu.SemaphoreType.DMA,
            pltpu.SemaphoreType.DMA,
            pltpu.SemaphoreType.DMA,
            pltpu.SemaphoreType.DMA,
            pltpu.SemaphoreType.DMA,
        ],
    )
    def sc_kernel(src_hbm, edge_hbm, sidx_hbm, didx_hbm,
                  out_a, out_b, out_deg,
                  acc, deg_acc, idx_s0, idx_s1, idx_d0, idx_d1,
                  rows0, rows1, ones, zvec, big_s, big_d,
                  idx_s2, idx_d2,
                  sem_g0, sem_g1, sem_s0, sem_s1, sem_h0, sem_h1):
        c = lax.axis_index("c")
        s = lax.axis_index("s")
        zeros16 = jnp.zeros((_L,), jnp.float32)
        ones16 = jnp.full((_L,), 1.0, jnp.float32)
        base_rows = s * rpt
        idx_s = [idx_s0, idx_s1]
        idx_d = [idx_d0, idx_d1]
        rows = [rows0, rows1]
        sem_g = [sem_g0, sem_g1]
        sem_s = [sem_s0, sem_s1]
        sem_h = [sem_h0, sem_h1]

        def zrow(r, _):
            for j in range(d // _L):
                rows0[r, pl.ds(j * _L, _L)] = zeros16
            return _
        lax.fori_loop(0, K, zrow, None)

        def vec_init(j, _):
            ones[pl.ds(j * _L, _L)] = ones16
            zvec[pl.ds(j * _L, _L)] = zeros16
            return _
        lax.fori_loop(0, K // _L, vec_init, None)

        def zero_slices(row0, cnt):
            off = 0
            while off < cnt:
                step = min(K, cnt - off)
                pltpu.sync_copy(rows0.at[pl.ds(0, step)],
                                acc.at[pl.ds(row0 + off, step)])
                pltpu.sync_copy(zvec.at[pl.ds(0, step)],
                                deg_acc.at[pl.ds(row0 + off, step)])
                off += step

        zero_slices(base_rows, rpt)
        if rem:
            @pl.when(s == _NS - 1)
            def _():
                zero_slices(_NS * rpt, rem)

        plsc.subcore_barrier()

        tile_base = s * ept

        def copy_idx(dst_ref, src_big, off):
            for j in range(K // _L):
                dst_ref[pl.ds(j * _L, _L)] = src_big[pl.ds(off + j * _L, _L)]

        def load_big(b):
            base = tile_base + b * G * K
            pltpu.sync_copy(didx_hbm.at[pl.ds(base, G * K)], big_d)

            @pl.when(c == 0)
            def _():
                pltpu.sync_copy(sidx_hbm.at[pl.ds(base, G * K)], big_s)

        def start_fetch(g, p):
            @pl.when(c == 0)
            def _():
                pltpu.async_copy(src_hbm.at[idx_s[p]], rows[p], sem_g[p])

            @pl.when(c == 1)
            def _():
                base = tile_base + g * K
                pltpu.async_copy(edge_hbm.at[pl.ds(base, K)], rows[p],
                                 sem_g[p])

        def wait_fetch(g, p):
            @pl.when(c == 0)
            def _():
                pltpu.make_async_copy(src_hbm.at[idx_s[p]], rows[p],
                                      sem_g[p]).wait()

            @pl.when(c == 1)
            def _():
                base = tile_base + g * K
                pltpu.make_async_copy(edge_hbm.at[pl.ds(base, K)], rows[p],
                                      sem_g[p]).wait()

        def start_scatter(p):
            pltpu.async_copy(rows[p], acc.at[idx_d[p]], sem_s[p], add=True)

            @pl.when(c == 1)
            def _():
                pltpu.async_copy(ones, deg_acc.at[idx_d[p]], sem_h[p],
                                 add=True)

        def wait_scatter(p):
            pltpu.make_async_copy(rows[p], acc.at[idx_d[p]], sem_s[p]).wait()

            @pl.when(c == 1)
            def _():
                pltpu.make_async_copy(ones, deg_acc.at[idx_d[p]],
                                      sem_h[p]).wait()

        def stage_idx(p, off):
            copy_idx(idx_d[p], big_d, off)

            @pl.when(c == 0)
            def _():
                copy_idx(idx_s[p], big_s, off)

        load_big(0)
        stage_idx(0, 0)
        start_fetch(0, 0)

        def block_body(b, _):
            for jj in range(G):
                g = b * G + jj
                p = jj % 2
                q = 1 - p
                wait_fetch(g, p)
                start_scatter(p)

                @pl.when(g > 0)
                def _():
                    wait_scatter(q)

                if jj < G - 1:
                    stage_idx(q, (jj + 1) * K)
                    start_fetch(g + 1, q)
                else:
                    @pl.when(b < n_blocks - 1)
                    def _():
                        load_big(b + 1)
                        stage_idx(q, 0)
                        start_fetch(g + 1, q)
            return _
        lax.fori_loop(0, n_blocks, block_body, None)
        wait_scatter((G - 1) % 2)

        if tail:
            base = tile_base + n_main * K
            tail_rows = rows0.at[pl.ds(0, tail)]
            pltpu.sync_copy(didx_hbm.at[pl.ds(base, tail)], idx_d2)

            @pl.when(c == 0)
            def _():
                pltpu.sync_copy(sidx_hbm.at[pl.ds(base, tail)], idx_s2)
                pltpu.async_copy(src_hbm.at[idx_s2], tail_rows,
                                 sem_g0).wait()
                pltpu.sync_copy(tail_rows, acc.at[idx_d2], add=True)

            @pl.when(c == 1)
            def _():
                pltpu.sync_copy(edge_hbm.at[pl.ds(base, tail)], tail_rows)
                pltpu.sync_copy(tail_rows, acc.at[idx_d2], add=True)
                pltpu.sync_copy(ones.at[pl.ds(0, tail)],
                                deg_acc.at[idx_d2], add=True)

        plsc.subcore_barrier()

        def drain_acc(dst_hbm, row0, cnt):
            off = 0
            while off < cnt:
                step = min(K, cnt - off)
                sl = pl.ds(row0 + off, step)
                pltpu.sync_copy(acc.at[sl], rows0.at[pl.ds(0, step)])
                pltpu.sync_copy(rows0.at[pl.ds(0, step)], dst_hbm.at[sl])
                off += step

        def drain_deg(row0, cnt):
            off = 0
            while off < cnt:
                step = min(K, cnt - off)
                sl = pl.ds(row0 + off, step)
                pltpu.sync_copy(deg_acc.at[sl], zvec.at[pl.ds(0, step)])
                pltpu.sync_copy(zvec.at[pl.ds(0, step)], out_deg.at[sl])
                off += step

        @pl.when(c == 0)
        def _():
            drain_acc(out_a, base_rows, rpt)
            if rem:
                @pl.when(s == _NS - 1)
                def _():
                    drain_acc(out_a, _NS * rpt, rem)

        @pl.when(c == 1)
        def _():
            drain_acc(out_b, base_rows, rpt)
            drain_deg(base_rows, rpt)
            if rem:
                @pl.when(s == _NS - 1)
                def _():
                    drain_acc(out_b, _NS * rpt, rem)
                    drain_deg(_NS * rpt, rem)

    return sc_kernel(src_feat, edge_feat, src_idx, dst_idx)


def _tc_finish(sa, sb, deg, w1, w2, bias):
    n, d = sa.shape
    bn = 1000
    assert n % bn == 0

    def body(a_ref, b_ref, deg_ref, w1_ref, w2_ref, bias_ref, out_ref):
        acc = jnp.dot(a_ref[...], w1_ref[...], preferred_element_type=jnp.float32)
        acc = acc + jnp.dot(b_ref[...], w2_ref[...], preferred_element_type=jnp.float32)
        dv = deg_ref[...]
        norm = jnp.where(dv > 0.0, lax.rsqrt(dv), 0.0)
        out_ref[...] = acc * norm + bias_ref[...]

    return pl.pallas_call(
        body,
        grid=(n // bn,),
        in_specs=[
            pl.BlockSpec((bn, d), lambda i: (i, 0)),
            pl.BlockSpec((bn, d), lambda i: (i, 0)),
            pl.BlockSpec((bn, 1), lambda i: (i, 0)),
            pl.BlockSpec((d, d), lambda i: (0, 0)),
            pl.BlockSpec((d, d), lambda i: (0, 0)),
            pl.BlockSpec((1, d), lambda i: (0, 0)),
        ],
        out_specs=pl.BlockSpec((bn, d), lambda i: (i, 0)),
        out_shape=jax.ShapeDtypeStruct((n, d), jnp.float32),
    )(sa, sb, deg, w1, w2, bias)


def kernel(src_feat, dst_feat, edge_feat, edge_index, W, b):
    del dst_feat
    n, d = src_feat.shape
    src_idx = edge_index[0]
    dst_idx = edge_index[1]
    sa, sb, deg = _sc_segment_sums(src_feat, edge_feat, src_idx, dst_idx)
    return _tc_finish(sa, sb, deg.reshape(n, 1), W[:d], W[d:], b.reshape(1, d))


# speedup vs baseline: 14.2603x; 1.0012x over previous
import functools

import jax
import jax.numpy as jnp
from jax import lax
from jax.experimental import pallas as pl
from jax.experimental.pallas import tpu as pltpu
from jax.experimental.pallas import tpu_sc as plsc

_NC = 2
_NS = 16
_L = 16


def _sc_segment_sums(src_feat, edge_feat, src_idx, dst_idx):
    n, d = src_feat.shape
    e = src_idx.shape[0]
    assert d == 128 and n % _NS == 0 and n % 8 == 0
    ept = e // _NS
    K = 128
    n_main = ept // K
    tail = ept - n_main * K
    assert tail % 8 == 0 and e % _NS == 0
    G = 6
    assert n_main % G == 0 and n_main % 2 == 0
    n_blocks = n_main // G
    rpt = (n // (8 * _NS)) * 8
    rem = n - _NS * rpt

    mesh = plsc.VectorSubcoreMesh(
        core_axis_name="c", subcore_axis_name="s",
        num_cores=_NC, num_subcores=_NS)

    @functools.partial(
        pl.kernel,
        out_type=[
            jax.ShapeDtypeStruct((n, d), jnp.float32),
            jax.ShapeDtypeStruct((n, d), jnp.float32),
            jax.ShapeDtypeStruct((n,), jnp.float32),
        ],
        mesh=mesh,
    scratch_types=[
            pltpu.VMEM_SHARED((n, d), jnp.float32),
            pltpu.VMEM_SHARED((n,), jnp.float32),
            pltpu.VMEM((K,), jnp.int32),
            pltpu.VMEM((K,), jnp.int32),
            pltpu.VMEM((K,), jnp.int32),
            pltpu.VMEM((K,), jnp.int32),
            pltpu.VMEM((K, d), jnp.float32),
            pltpu.VMEM((K, d), jnp.float32),
            pltpu.VMEM((K,), jnp.float32),
            pltpu.VMEM((K,), jnp.float32),
            pltpu.VMEM((G * K,), jnp.int32),
            pltpu.VMEM((G * K,), jnp.int32),
            pltpu.VMEM((tail,), jnp.int32),
            pltpu.VMEM((tail,), jnp.int32),
            pltpu.SemaphoreType.DMA,
            pltpu.SemaphoreType.DMA,
            pltpu.SemaphoreType.DMA,
            pltpu.SemaphoreType.DMA,
            pltpu.SemaphoreType.DMA,
            pltpu.SemaphoreType.DMA,
        ],
    )
    def sc_kernel(src_hbm, edge_hbm, sidx_hbm, didx_hbm,
                  out_a, out_b, out_deg,
                  acc, deg_acc, idx_s0, idx_s1, idx_d0, idx_d1,
                  rows0, rows1, ones, zvec, big_s, big_d,
                  idx_s2, idx_d2,
                  sem_g0, sem_g1, sem_s0, sem_s1, sem_h0, sem_h1):
        c = lax.axis_index("c")
        s = lax.axis_index("s")
        zeros16 = jnp.zeros((_L,), jnp.float32)
        ones16 = jnp.full((_L,), 1.0, jnp.float32)
        base_rows = s * rpt
        idx_s = [idx_s0, idx_s1]
        idx_d = [idx_d0, idx_d1]
        rows = [rows0, rows1]
        sem_g = [sem_g0, sem_g1]
        sem_s = [sem_s0, sem_s1]
        sem_h = [sem_h0, sem_h1]

        def zrow(r, _):
            for j in range(d // _L):
                rows0[r, pl.ds(j * _L, _L)] = zeros16
            return _
        lax.fori_loop(0, K, zrow, None)

        def vec_init(j, _):
            ones[pl.ds(j * _L, _L)] = ones16
            zvec[pl.ds(j * _L, _L)] = zeros16
            return _
        lax.fori_loop(0, K // _L, vec_init, None)

        def zero_slices(row0, cnt):
            off = 0
            while off < cnt:
                step = min(K, cnt - off)
                pltpu.sync_copy(rows0.at[pl.ds(0, step)],
                                acc.at[pl.ds(row0 + off, step)])
                pltpu.sync_copy(zvec.at[pl.ds(0, step)],
                                deg_acc.at[pl.ds(row0 + off, step)])
                off += step

        zero_slices(base_rows, rpt)
        if rem:
            @pl.when(s == _NS - 1)
            def _():
                zero_slices(_NS * rpt, rem)

        plsc.subcore_barrier()

        tile_base = s * ept

        def copy_idx(dst_ref, src_big, off):
            for j in range(K // _L):
                dst_ref[pl.ds(j * _L, _L)] = src_big[pl.ds(off + j * _L, _L)]

        def load_big(b):
            base = tile_base + b * G * K
            pltpu.sync_copy(didx_hbm.at[pl.ds(base, G * K)], big_d)

            @pl.when(c == 0)
            def _():
                pltpu.sync_copy(sidx_hbm.at[pl.ds(base, G * K)], big_s)

        def start_fetch(g, p):
            @pl.when(c == 0)
            def _():
                pltpu.async_copy(src_hbm.at[idx_s[p]], rows[p], sem_g[p])

            @pl.when(c == 1)
            def _():
                base = tile_base + g * K
                pltpu.async_copy(edge_hbm.at[pl.ds(base, K)], rows[p],
                                 sem_g[p])

        def wait_fetch(g, p):
            @pl.when(c == 0)
            def _():
                pltpu.make_async_copy(src_hbm.at[idx_s[p]], rows[p],
                                      sem_g[p]).wait()

            @pl.when(c == 1)
            def _():
                base = tile_base + g * K
                pltpu.make_async_copy(edge_hbm.at[pl.ds(base, K)], rows[p],
                                      sem_g[p]).wait()

        def start_scatter(p):
            pltpu.async_copy(rows[p], acc.at[idx_d[p]], sem_s[p], add=True)

            @pl.when(c == 1)
            def _():
                pltpu.async_copy(ones, deg_acc.at[idx_d[p]], sem_h[p],
                                 add=True)

        def wait_scatter(p):
            pltpu.make_async_copy(rows[p], acc.at[idx_d[p]], sem_s[p]).wait()

            @pl.when(c == 1)
            def _():
                pltpu.make_async_copy(ones, deg_acc.at[idx_d[p]],
                                      sem_h[p]).wait()

        def stage_idx(p, off):
            copy_idx(idx_d[p], big_d, off)

            @pl.when(c == 0)
            def _():
                copy_idx(idx_s[p], big_s, off)

        load_big(0)
        stage_idx(0, 0)
        start_fetch(0, 0)

        def block_body(b, _):
            for jj in range(G):
                g = b * G + jj
                p = jj % 2
                q = 1 - p
                wait_fetch(g, p)
                start_scatter(p)

                @pl.when(g > 0)
                def _():
                    wait_scatter(q)

                if jj < G - 1:
                    stage_idx(q, (jj + 1) * K)
                    start_fetch(g + 1, q)
                else:
                    @pl.when(b < n_blocks - 1)
                    def _():
                        load_big(b + 1)
                        stage_idx(q, 0)
                        start_fetch(g + 1, q)
            return _
        lax.fori_loop(0, n_blocks, block_body, None)
        wait_scatter((G - 1) % 2)

        if tail:
            base = tile_base + n_main * K
            tail_rows = rows0.at[pl.ds(0, tail)]
            pltpu.sync_copy(didx_hbm.at[pl.ds(base, tail)], idx_d2)

            @pl.when(c == 0)
            def _():
                pltpu.sync_copy(sidx_hbm.at[pl.ds(base, tail)], idx_s2)
                pltpu.async_copy(src_hbm.at[idx_s2], tail_rows,
                                 sem_g0).wait()
                pltpu.sync_copy(tail_rows, acc.at[idx_d2], add=True)

            @pl.when(c == 1)
            def _():
                pltpu.sync_copy(edge_hbm.at[pl.ds(base, tail)], tail_rows)
                pltpu.sync_copy(tail_rows, acc.at[idx_d2], add=True)
                pltpu.sync_copy(ones.at[pl.ds(0, tail)],
                                deg_acc.at[idx_d2], add=True)

        plsc.subcore_barrier()

        def drain_acc(dst_hbm, row0, cnt):
            sl = pl.ds(row0, cnt)
            pltpu.sync_copy(acc.at[sl], dst_hbm.at[sl])

        def drain_deg(row0, cnt):
            off = 0
            while off < cnt:
                step = min(K, cnt - off)
                sl = pl.ds(row0 + off, step)
                pltpu.sync_copy(deg_acc.at[sl], zvec.at[pl.ds(0, step)])
                pltpu.sync_copy(zvec.at[pl.ds(0, step)], out_deg.at[sl])
                off += step

        @pl.when(c == 0)
        def _():
            drain_acc(out_a, base_rows, rpt)
            if rem:
                @pl.when(s == _NS - 1)
                def _():
                    drain_acc(out_a, _NS * rpt, rem)

        @pl.when(c == 1)
        def _():
            drain_acc(out_b, base_rows, rpt)
            drain_deg(base_rows, rpt)
            if rem:
                @pl.when(s == _NS - 1)
                def _():
                    drain_acc(out_b, _NS * rpt, rem)
                    drain_deg(_NS * rpt, rem)

    return sc_kernel(src_feat, edge_feat, src_idx, dst_idx)


def _tc_finish(sa, sb, deg, w1, w2, bias):
    n, d = sa.shape
    bn = 1000
    assert n % bn == 0

    def body(a_ref, b_ref, deg_ref, w1_ref, w2_ref, bias_ref, out_ref):
        acc = jnp.dot(a_ref[...], w1_ref[...], preferred_element_type=jnp.float32)
        acc = acc + jnp.dot(b_ref[...], w2_ref[...], preferred_element_type=jnp.float32)
        dv = deg_ref[...]
        norm = jnp.where(dv > 0.0, lax.rsqrt(dv), 0.0)
        out_ref[...] = acc * norm + bias_ref[...]

    return pl.pallas_call(
        body,
        grid=(n // bn,),
        in_specs=[
            pl.BlockSpec((bn, d), lambda i: (i, 0)),
            pl.BlockSpec((bn, d), lambda i: (i, 0)),
            pl.BlockSpec((bn, 1), lambda i: (i, 0)),
            pl.BlockSpec((d, d), lambda i: (0, 0)),
            pl.BlockSpec((d, d), lambda i: (0, 0)),
            pl.BlockSpec((1, d), lambda i: (0, 0)),
        ],
        out_specs=pl.BlockSpec((bn, d), lambda i: (i, 0)),
        out_shape=jax.ShapeDtypeStruct((n, d), jnp.float32),
    )(sa, sb, deg, w1, w2, bias)


def kernel(src_feat, dst_feat, edge_feat, edge_index, W, b):
    del dst_feat
    n, d = src_feat.shape
    src_idx = edge_index[0]
    dst_idx = edge_index[1]
    sa, sb, deg = _sc_segment_sums(src_feat, edge_feat, src_idx, dst_idx)
    return _tc_finish(sa, sb, deg.reshape(n, 1), W[:d], W[d:], b.reshape(1, d))
